# baseline (speedup 1.0000x reference)
import functools

import jax
import jax.numpy as jnp
from jax import lax
from jax.experimental import pallas as pl
from jax.experimental.pallas import tpu as pltpu
from jax.experimental.pallas import tpu_sc as plsc
import numpy as np

N = 10000
E = 160000
EXTENT = np.float32(1.5 * 6 * 0.025)
NSUB = 32
SB = 1024
NSTG = 5
EPS = SB * NSTG
EPAD = NSUB * EPS
CG = 64
SBC = 512
NPAIR = SBC // (2 * CG)
EPSR = EPS + SB
NSTGR = EPSR // SBC
NPAD = 10240
RPS = NPAD // 16

_MESH = dict(mesh=plsc.VectorSubcoreMesh(core_axis_name="c", subcore_axis_name="s"),
             compiler_params=pltpu.CompilerParams(needs_layout_passes=False,
                                                  use_tc_tiling_on_sc=False))

def _wid():
    return lax.axis_index("s") * 2 + lax.axis_index("c")


def _splat_i32(v):
    return (jnp.broadcast_to(jnp.int32(v), (16,)) if isinstance(v, int)
            else lax.broadcast(v, (16,)))


def _sqrt16(x):
    i = plsc.bitcast(x, jnp.int32)
    y = plsc.bitcast(lax.shift_right_logical(i, 1) + jnp.int32(0x1FBD1DF5), jnp.float32)
    for _ in range(3):
        y = 0.5 * (y + x / y)
    return y


def _geom_body(sx_h, sy_h, sz_h, dx_h, dy_h, dz_h, src_h, dst_h,
               w_out, i_out, d_out, c_out,
               sx, sy, sz, dx, dy, dz, srcv, dstv, wbuf, ibuf, dbuf, cntbuf):
    wid = _wid()
    pltpu.sync_copy(sx_h, sx)
    pltpu.sync_copy(sy_h, sy)
    pltpu.sync_copy(sz_h, sz)
    pltpu.sync_copy(dx_h, dx)
    pltpu.sync_copy(dy_h, dy)
    pltpu.sync_copy(dz_h, dz)
    scale = jnp.float32(2.0 / EXTENT)
    zf = jnp.zeros((16,), jnp.float32)
    zi = jnp.zeros((16,), jnp.int32)
    cnt = jnp.int32(0)

    for kc in range(NSTG):
        base = wid * EPS + kc * SB
        pltpu.sync_copy(src_h.at[pl.ds(base, SB)], srcv)
        pltpu.sync_copy(dst_h.at[pl.ds(base, SB)], dstv)

        def zrow(r, _):
            for c in range(8):
                wbuf[c, pl.ds(r * 16, 16)] = zf
                ibuf[c, pl.ds(r * 16, 16)] = zi
            dbuf[pl.ds(r * 16, 16)] = zi
            return _
        lax.fori_loop(0, (SB + 16) // 16, zrow, None)

        def grp(g, off):
            s16 = srcv[pl.ds(g * 16, 16)]
            d16 = dstv[pl.ds(g * 16, 16)]
            xs = plsc.load_gather(sx, [s16])
            ys = plsc.load_gather(sy, [s16])
            zs = plsc.load_gather(sz, [s16])
            xd = plsc.load_gather(dx, [d16])
            yd = plsc.load_gather(dy, [d16])
            zd = plsc.load_gather(dz, [d16])
            rx = (xs - xd) * scale
            ry = (ys - yd) * scale
            rz = (zs - zd) * scale
            r2 = rx * rx + ry * ry + rz * rz
            t = 1.0 - r2
            win = jnp.clip(t * t * t, 0.0, 1.0)
            eid = base + g * 16 + lax.iota(jnp.int32, 16)
            win = jnp.where(eid < E, win, 0.0)
            mask = win > 0.0
            l2 = _sqrt16(jnp.maximum(r2, 1e-12))
            linf = jnp.maximum(
                jnp.maximum(jnp.maximum(jnp.abs(rx), jnp.abs(ry)), jnp.abs(rz)),
                1e-12)
            q = l2 / linf
            ax = jnp.clip(rx * q, -1.0, 1.0)
            ay = jnp.clip(ry * q, -1.0, 1.0)
            az = jnp.clip(rz * q, -1.0, 1.0)
            ux = (ax + 1.0) * 1.5
            uy = (ay + 1.0) * 1.5
            uz = (az + 1.0) * 1.5
            lx = jnp.minimum(ux.astype(jnp.int32), 2)
            ly = jnp.minimum(uy.astype(jnp.int32), 2)
            lz = jnp.minimum(uz.astype(jnp.int32), 2)
            fx = ux - lx.astype(jnp.float32)
            fy = uy - ly.astype(jnp.float32)
            fz = uz - lz.astype(jnp.float32)
            base_i = s16 * 64 + lx * 16 + ly * 4 + lz
            for c in range(8):
                ox, oy, oz = (c >> 2) & 1, (c >> 1) & 1, c & 1
                w = ((fx if ox else 1.0 - fx)
                     * (fy if oy else 1.0 - fy)
                     * (fz if oz else 1.0 - fz) * win)
                gi = base_i + (ox * 16 + oy * 4 + oz)
                plsc.store_compressed(wbuf.at[c, pl.ds(off, 16)], w, mask=mask)
                plsc.store_compressed(ibuf.at[c, pl.ds(off, 16)], gi, mask=mask)
            plsc.store_compressed(dbuf.at[pl.ds(off, 16)], d16, mask=mask)
            return off + jnp.sum(jnp.where(mask, 1, 0).astype(jnp.int32))

        off = lax.fori_loop(0, SB // 16, grp, jnp.int32(0))
        cnt = pl.multiple_of(cnt, 8)
        pltpu.sync_copy(wbuf.at[:, pl.ds(0, SB)],
                        w_out.at[:, pl.ds(wid * EPSR + cnt, SB)])
        pltpu.sync_copy(ibuf.at[:, pl.ds(0, SB)],
                        i_out.at[:, pl.ds(wid * EPSR + cnt, SB)])
        pltpu.sync_copy(dbuf.at[pl.ds(0, SB)],
                        d_out.at[pl.ds(wid * EPSR + cnt, SB)])
        cnt = cnt + jnp.bitwise_and(off + 7, jnp.int32(-8))

    cnt = pl.multiple_of(cnt, 8)
    def zrow2(r, _):
        for c in range(8):
            wbuf[c, pl.ds(r * 16, 16)] = zf
            ibuf[c, pl.ds(r * 16, 16)] = zi
        dbuf[pl.ds(r * 16, 16)] = zi
        return _
    lax.fori_loop(0, (SB + 16) // 16, zrow2, None)
    pltpu.sync_copy(wbuf.at[:, pl.ds(0, SB)],
                    w_out.at[:, pl.ds(wid * EPSR + cnt, SB)])
    pltpu.sync_copy(ibuf.at[:, pl.ds(0, SB)],
                    i_out.at[:, pl.ds(wid * EPSR + cnt, SB)])
    pltpu.sync_copy(dbuf.at[pl.ds(0, SB)],
                    d_out.at[pl.ds(wid * EPSR + cnt, SB)])

    cntbuf[pl.ds(0, 16)] = lax.broadcast(cnt, (16,))
    pltpu.sync_copy(cntbuf.at[pl.ds(0, 8)], c_out.at[wid])


_geom = functools.partial(
    pl.kernel,
    out_type=[jax.ShapeDtypeStruct((8, NSUB * EPSR), jnp.float32),
              jax.ShapeDtypeStruct((8, NSUB * EPSR), jnp.int32),
              jax.ShapeDtypeStruct((NSUB * EPSR,), jnp.int32),
              jax.ShapeDtypeStruct((NSUB, 8), jnp.int32)],
    scratch_types=[pltpu.VMEM((N,), jnp.float32)] * 6
    + [pltpu.VMEM((SB,), jnp.int32)] * 2
    + [pltpu.VMEM((8, SB + 16), jnp.float32), pltpu.VMEM((8, SB + 16), jnp.int32),
       pltpu.VMEM((SB + 16,), jnp.int32), pltpu.VMEM((16,), jnp.int32)],
    **_MESH,
)(_geom_body)


def _conv_body(cout, cgs, y, w8, i8, dst_h, cnt_h, out,
               wv, iv, dstv, cntv, rows_a, rows_b, acc_a, acc_b, zbuf, out_sh,
               sem_a, sem_b, sem_sa, sem_sb):
    wid = _wid()
    scid = lax.axis_index("c")
    sid = lax.axis_index("s")
    nq = cout // 16

    def zrow(r, _):
        for qq in range(nq):
            zbuf[r, pl.ds(qq * 16, 16)] = jnp.zeros((16,), jnp.float32)
        return _
    lax.fori_loop(0, 64, zrow, None)
    for j in range(10):
        pltpu.sync_copy(zbuf, out_sh.at[pl.ds(sid * RPS + j * 64, 64)])
    plsc.subcore_barrier()

    def fire(cg, rows, sem):
        for c in range(8):
            pltpu.async_copy(y.at[iv.at[c, pl.ds(cg * cgs, cgs)]], rows.at[c], sem)

    def drain(cg, rows, sem):
        for c in range(8):
            pltpu.make_async_copy(y.at[iv.at[c, pl.ds(cg * cgs, cgs)]],
                                  rows.at[c], sem).wait()

    def compute(cg, rows, acc):
        def grp(g, _):
            e0 = g * 16
            for j in range(16):
                e = e0 + j
                ev = _splat_i32(cg * cgs + e)
                av = [None] * nq
                for c in range(8):
                    ws = plsc.load_gather(wv, [_splat_i32(c), ev])
                    for qq in range(nq):
                        term = ws * rows[c, e, pl.ds(qq * 16, 16)]
                        av[qq] = term if c == 0 else av[qq] + term
                for qq in range(nq):
                    acc[e, pl.ds(qq * 16, 16)] = av[qq]
            return _
        lax.fori_loop(0, cgs // 16, grp, None)

    def scatter_fire(cg, acc, sem):
        pltpu.async_copy(acc, out_sh.at[dstv.at[cg]], sem, add=True)

    def scatter_wait(acc, sem):
        pltpu.make_async_copy(acc, out_sh.at[dstv.at[0]], sem).wait()

    pltpu.sync_copy(cnt_h.at[wid], cntv.at[pl.ds(0, 8)])
    cnt = cntv[pl.ds(0, 16)][0]
    nst = (cnt + (SBC - 1)) // SBC

    def stage(s, _):
        base = wid * EPSR + s * SBC
        sb0 = s * SBC
        pltpu.sync_copy(w8.at[:, pl.ds(base, SBC)], wv)
        pltpu.sync_copy(i8.at[:, pl.ds(base, SBC)], iv)
        pltpu.sync_copy(dst_h.at[pl.ds(base // cgs, SBC // cgs)], dstv)
        fire(0, rows_a, sem_a)

        def pair(p, _):
            ca = 2 * p
            cb = 2 * p + 1
            act_a = sb0 + ca * cgs < cnt
            act_b = sb0 + cb * cgs < cnt

            @pl.when(act_b)
            def _():
                fire(cb, rows_b, sem_b)

            @pl.when(jnp.logical_and(act_a, p > 0))
            def _():
                scatter_wait(acc_a, sem_sa)

            @pl.when(act_a)
            def _():
                drain(ca, rows_a, sem_a)
                compute(ca, rows_a, acc_a)
                scatter_fire(ca, acc_a, sem_sa)

            @pl.when(jnp.logical_and(p < SBC // (2 * cgs) - 1, sb0 + (ca + 2) * cgs < cnt))
            def _():
                fire(ca + 2, rows_a, sem_a)

            @pl.when(jnp.logical_and(act_b, p > 0))
            def _():
                scatter_wait(acc_b, sem_sb)

            @pl.when(act_b)
            def _():
                drain(cb, rows_b, sem_b)
                compute(cb, rows_b, acc_b)
                scatter_fire(cb, acc_b, sem_sb)
            return _

        lax.fori_loop(0, SBC // (2 * cgs), pair, None)
        scatter_wait(acc_a, sem_sa)

        @pl.when(sb0 + cgs < cnt)
        def _():
            scatter_wait(acc_b, sem_sb)
        return _

    lax.fori_loop(0, nst, stage, None)
    plsc.subcore_barrier()
    for j in range(5):
        r0 = sid * RPS + j * 128
        pltpu.sync_copy(out_sh.at[pl.ds(r0, 128)], out.at[scid, pl.ds(r0, 128)])


def _conv(y, w8, i8, dst_h, cnt_h, cout, cgs):
    body = functools.partial(_conv_body, cout, cgs)
    return pl.kernel(
        body,
        out_type=jax.ShapeDtypeStruct((2, NPAD, cout), jnp.float32),
        scratch_types=[
            pltpu.VMEM((8, SBC), jnp.float32),
            pltpu.VMEM((8, SBC), jnp.int32),
            pltpu.VMEM((SBC // cgs, cgs), jnp.int32),
            pltpu.VMEM((16,), jnp.int32),
            pltpu.VMEM((8, cgs, cout), jnp.float32),
            pltpu.VMEM((8, cgs, cout), jnp.float32),
            pltpu.VMEM((cgs, cout), jnp.float32),
            pltpu.VMEM((cgs, cout), jnp.float32),
            pltpu.VMEM((64, cout), jnp.float32),
            pltpu.VMEM_SHARED((NPAD, cout), jnp.float32),
            pltpu.SemaphoreType.DMA,
            pltpu.SemaphoreType.DMA,
            pltpu.SemaphoreType.DMA,
            pltpu.SemaphoreType.DMA,
        ],
        **_MESH,
    )(y, w8, i8, dst_h, cnt_h)


def _mm(x, w, b=None, relu=False):
    m, kin = x.shape
    kout = w.shape[1]
    bn = 1000
    bk = min(kout, 512)

    def body(*refs):
        x_ref, w_ref = refs[0], refs[1]
        o_ref = refs[-1]
        xx = x_ref[...]
        if relu:
            xx = jnp.maximum(xx, 0.0)
        o = jnp.dot(xx, w_ref[...], preferred_element_type=jnp.float32)
        if b is not None:
            o = o + refs[2][...]
        o_ref[...] = o

    in_specs = [pl.BlockSpec((bn, kin), lambda i, j: (i, 0)),
                pl.BlockSpec((kin, bk), lambda i, j: (0, j))]
    args = [x, w]
    if b is not None:
        in_specs.append(pl.BlockSpec((1, bk), lambda i, j: (0, j)))
        args.append(b.reshape(1, kout))
    return pl.pallas_call(
        body,
        grid=(m // bn, kout // bk),
        in_specs=in_specs,
        out_specs=pl.BlockSpec((bn, bk), lambda i, j: (i, j)),
        out_shape=jax.ShapeDtypeStruct((m, kout), jnp.float32),
    )(*args)


def _addk(arrays, scale=1.0):
    m, c = arrays[0].shape
    bn = 1000

    def body(*refs):
        o_ref = refs[-1]
        s = refs[0][...]
        for r in refs[1:-1]:
            s = s + r[...]
        o_ref[...] = s * scale

    return pl.pallas_call(
        body,
        grid=(m // bn,),
        in_specs=[pl.BlockSpec((bn, c), lambda i: (i, 0)) for _ in arrays],
        out_specs=pl.BlockSpec((bn, c), lambda i: (i, 0)),
        out_shape=jax.ShapeDtypeStruct((m, c), jnp.float32),
    )(*arrays)


def _kflat(kk, cout_pad=None):
    cin, cout = kk.shape[3], kk.shape[4]
    k = kk.reshape(64, cin, cout)
    if cout_pad is not None and cout_pad != cout:
        k = jnp.pad(k, ((0, 0), (0, 0), (0, cout_pad - cout)))
        cout = cout_pad
    return k.transpose(1, 0, 2).reshape(cin, 64 * cout)


def kernel(pos, vel, box, box_feats, edge_index_fluid, edge_index_obstacle,
           k0f, k0o, d0W, d0b, k1, d1W, d1b, k2, d2W, d2b, k3, d3W, d3b):
    px, py, pz = pos[:, 0], pos[:, 1], pos[:, 2]
    bx, by, bz = box[:, 0], box[:, 1], box[:, 2]
    ep = EPAD - E
    srcf = jnp.pad(edge_index_fluid[0], (0, ep))
    dstf = jnp.pad(edge_index_fluid[1], (0, ep))
    srco = jnp.pad(edge_index_obstacle[0], (0, ep))
    dsto = jnp.pad(edge_index_obstacle[1], (0, ep))
    wf8, if8, dcf, ccf = _geom(px, py, pz, px, py, pz, srcf, dstf)
    wo8, io8, dco, cco = _geom(bx, by, bz, px, py, pz, srco, dsto)

    fluid_feats = jnp.concatenate([jnp.ones_like(pos[:, 0:1]), vel], axis=-1)
    ff_pad = jnp.pad(fluid_feats, ((0, 0), (0, 4)))
    bf_pad = jnp.pad(box_feats, ((0, 0), (0, 5)))

    def conv(x_pad, kflat_mat, w8, i8, dst_h, cnt_h, cout, relu):
        cgs = 64 if cout == 64 else 128
        p = _conv(_mm(x_pad, kflat_mat, relu=relu).reshape(N * 64, cout),
                  w8, i8, dst_h.reshape(NSUB * EPSR // cgs, cgs), cnt_h,
                  cout, cgs)
        return p[0, :N], p[1, :N]

    k0f_f = _kflat(jnp.pad(k0f, ((0, 0),) * 3 + ((0, 4), (0, 0))))
    k0o_f = _kflat(jnp.pad(k0o, ((0, 0),) * 3 + ((0, 5), (0, 0))))
    c0f = _addk(conv(ff_pad, k0f_f, wf8, if8, dcf, ccf, 32, False))
    c0o = _addk(conv(bf_pad, k0o_f, wo8, io8, dco, cco, 32, False))
    d0 = _mm(ff_pad, jnp.pad(d0W, ((0, 4), (0, 0))), d0b, False)
    feats = jnp.concatenate([c0o, c0f, d0], axis=-1)

    p0, p1 = conv(feats, _kflat(k1), wf8, if8, dcf, ccf, 64, True)
    d1 = _mm(feats, d1W, d1b, True)
    ans1 = _addk((p0, p1, d1))
    p0, p1 = conv(ans1, _kflat(k2), wf8, if8, dcf, ccf, 64, True)
    d2 = _mm(ans1, d2W, d2b, True)
    ans2 = _addk((p0, p1, d2, ans1))
    p0, p1 = conv(ans2, _kflat(k3, cout_pad=16), wf8, if8, dcf, ccf, 16, True)
    d3 = _mm(ans2, jnp.pad(d3W, ((0, 0), (0, 13))),
             jnp.pad(d3b, (0, 13)), True)
    ans3 = _addk((p0, p1, d3), scale=1.0 / 128.0)
    return ans3[:, :3]


# speedup vs baseline: 1.1322x; 1.1322x over previous
import functools

import jax
import jax.numpy as jnp
from jax import lax
from jax.experimental import pallas as pl
from jax.experimental.pallas import tpu as pltpu
from jax.experimental.pallas import tpu_sc as plsc
import numpy as np

N = 10000
E = 160000
EXTENT = np.float32(1.5 * 6 * 0.025)
NSUB = 32
SB = 1024
NSTG = 5
EPS = SB * NSTG
EPAD = NSUB * EPS
CG = 64
SBC = 512
NPAIR = SBC // (2 * CG)
EPSR = EPS + SB
NSTGR = EPSR // SBC
NPAD = 10240
RPS = NPAD // 16

_MESH = dict(mesh=plsc.VectorSubcoreMesh(core_axis_name="c", subcore_axis_name="s"),
             compiler_params=pltpu.CompilerParams(needs_layout_passes=False,
                                                  use_tc_tiling_on_sc=False))

def _wid():
    return lax.axis_index("s") * 2 + lax.axis_index("c")


def _splat_i32(v):
    return (jnp.broadcast_to(jnp.int32(v), (16,)) if isinstance(v, int)
            else lax.broadcast(v, (16,)))


def _sqrt16(x):
    i = plsc.bitcast(x, jnp.int32)
    y = plsc.bitcast(lax.shift_right_logical(i, 1) + jnp.int32(0x1FBD1DF5), jnp.float32)
    for _ in range(3):
        y = 0.5 * (y + x / y)
    return y


def _geom_body(sx_h, sy_h, sz_h, dx_h, dy_h, dz_h, src_h, dst_h,
               w_out, i_out, d_out, c_out,
               sx, sy, sz, dx, dy, dz, srcv, dstv, wbuf, ibuf, dbuf, cntbuf):
    wid = _wid()
    pltpu.sync_copy(sx_h, sx)
    pltpu.sync_copy(sy_h, sy)
    pltpu.sync_copy(sz_h, sz)
    pltpu.sync_copy(dx_h, dx)
    pltpu.sync_copy(dy_h, dy)
    pltpu.sync_copy(dz_h, dz)
    scale = jnp.float32(2.0 / EXTENT)
    zf = jnp.zeros((16,), jnp.float32)
    zi = jnp.zeros((16,), jnp.int32)
    cnt = jnp.int32(0)

    for kc in range(NSTG):
        base = wid * EPS + kc * SB
        pltpu.sync_copy(src_h.at[pl.ds(base, SB)], srcv)
        pltpu.sync_copy(dst_h.at[pl.ds(base, SB)], dstv)

        def zrow(r, _):
            for c in range(8):
                wbuf[c, pl.ds(r * 16, 16)] = zf
                ibuf[c, pl.ds(r * 16, 16)] = zi
            dbuf[pl.ds(r * 16, 16)] = zi
            return _
        lax.fori_loop(0, (SB + 16) // 16, zrow, None)

        def grp(g, off):
            s16 = srcv[pl.ds(g * 16, 16)]
            d16 = dstv[pl.ds(g * 16, 16)]
            xs = plsc.load_gather(sx, [s16])
            ys = plsc.load_gather(sy, [s16])
            zs = plsc.load_gather(sz, [s16])
            xd = plsc.load_gather(dx, [d16])
            yd = plsc.load_gather(dy, [d16])
            zd = plsc.load_gather(dz, [d16])
            rx = (xs - xd) * scale
            ry = (ys - yd) * scale
            rz = (zs - zd) * scale
            r2 = rx * rx + ry * ry + rz * rz
            t = 1.0 - r2
            win = jnp.clip(t * t * t, 0.0, 1.0)
            eid = base + g * 16 + lax.iota(jnp.int32, 16)
            win = jnp.where(eid < E, win, 0.0)
            mask = win > 0.0
            l2 = _sqrt16(jnp.maximum(r2, 1e-12))
            linf = jnp.maximum(
                jnp.maximum(jnp.maximum(jnp.abs(rx), jnp.abs(ry)), jnp.abs(rz)),
                1e-12)
            q = l2 / linf
            ax = jnp.clip(rx * q, -1.0, 1.0)
            ay = jnp.clip(ry * q, -1.0, 1.0)
            az = jnp.clip(rz * q, -1.0, 1.0)
            ux = (ax + 1.0) * 1.5
            uy = (ay + 1.0) * 1.5
            uz = (az + 1.0) * 1.5
            lx = jnp.minimum(ux.astype(jnp.int32), 2)
            ly = jnp.minimum(uy.astype(jnp.int32), 2)
            lz = jnp.minimum(uz.astype(jnp.int32), 2)
            fx = ux - lx.astype(jnp.float32)
            fy = uy - ly.astype(jnp.float32)
            fz = uz - lz.astype(jnp.float32)
            base_i = s16 * 64 + lx * 16 + ly * 4 + lz
            for c in range(8):
                ox, oy, oz = (c >> 2) & 1, (c >> 1) & 1, c & 1
                w = ((fx if ox else 1.0 - fx)
                     * (fy if oy else 1.0 - fy)
                     * (fz if oz else 1.0 - fz) * win)
                gi = base_i + (ox * 16 + oy * 4 + oz)
                plsc.store_compressed(wbuf.at[c, pl.ds(off, 16)], w, mask=mask)
                plsc.store_compressed(ibuf.at[c, pl.ds(off, 16)], gi, mask=mask)
            plsc.store_compressed(dbuf.at[pl.ds(off, 16)], d16, mask=mask)
            return off + jnp.sum(jnp.where(mask, 1, 0).astype(jnp.int32))

        off = lax.fori_loop(0, SB // 16, grp, jnp.int32(0))
        cnt = pl.multiple_of(cnt, 8)
        pltpu.sync_copy(wbuf.at[:, pl.ds(0, SB)],
                        w_out.at[:, pl.ds(wid * EPSR + cnt, SB)])
        pltpu.sync_copy(ibuf.at[:, pl.ds(0, SB)],
                        i_out.at[:, pl.ds(wid * EPSR + cnt, SB)])
        pltpu.sync_copy(dbuf.at[pl.ds(0, SB)],
                        d_out.at[pl.ds(wid * EPSR + cnt, SB)])
        cnt = cnt + jnp.bitwise_and(off + 7, jnp.int32(-8))

    cnt = pl.multiple_of(cnt, 8)
    def zrow2(r, _):
        for c in range(8):
            wbuf[c, pl.ds(r * 16, 16)] = zf
            ibuf[c, pl.ds(r * 16, 16)] = zi
        dbuf[pl.ds(r * 16, 16)] = zi
        return _
    lax.fori_loop(0, (SB + 16) // 16, zrow2, None)
    pltpu.sync_copy(wbuf.at[:, pl.ds(0, SB)],
                    w_out.at[:, pl.ds(wid * EPSR + cnt, SB)])
    pltpu.sync_copy(ibuf.at[:, pl.ds(0, SB)],
                    i_out.at[:, pl.ds(wid * EPSR + cnt, SB)])
    pltpu.sync_copy(dbuf.at[pl.ds(0, SB)],
                    d_out.at[pl.ds(wid * EPSR + cnt, SB)])

    cntbuf[pl.ds(0, 16)] = lax.broadcast(cnt, (16,))
    pltpu.sync_copy(cntbuf.at[pl.ds(0, 8)], c_out.at[wid])


_geom = functools.partial(
    pl.kernel,
    out_type=[jax.ShapeDtypeStruct((8, NSUB * EPSR), jnp.float32),
              jax.ShapeDtypeStruct((8, NSUB * EPSR), jnp.int32),
              jax.ShapeDtypeStruct((NSUB * EPSR,), jnp.int32),
              jax.ShapeDtypeStruct((NSUB, 8), jnp.int32)],
    scratch_types=[pltpu.VMEM((N,), jnp.float32)] * 6
    + [pltpu.VMEM((SB,), jnp.int32)] * 2
    + [pltpu.VMEM((8, SB + 16), jnp.float32), pltpu.VMEM((8, SB + 16), jnp.int32),
       pltpu.VMEM((SB + 16,), jnp.int32), pltpu.VMEM((16,), jnp.int32)],
    **_MESH,
)(_geom_body)


def _conv_body(cout, cgs, y, w8, i8, dst_h, cnt_h, out,
               wv, iv, dstv, cntv, rows_a, rows_b, acc_a, acc_b, zbuf, out_sh,
               sem_a, sem_b, sem_sa, sem_sb):
    wid = _wid()
    scid = lax.axis_index("c")
    sid = lax.axis_index("s")
    nq = cout // 16

    def zrow(r, _):
        for qq in range(nq):
            zbuf[r, pl.ds(qq * 16, 16)] = jnp.zeros((16,), jnp.float32)
        return _
    lax.fori_loop(0, 64, zrow, None)
    for j in range(10):
        pltpu.sync_copy(zbuf, out_sh.at[pl.ds(sid * RPS + j * 64, 64)])
    plsc.subcore_barrier()

    def fire(cg, rows, sem):
        for c in range(8):
            pltpu.async_copy(y.at[iv.at[c, pl.ds(cg * cgs, cgs)]], rows.at[c], sem)

    def drain(cg, rows, sem):
        for c in range(8):
            pltpu.make_async_copy(y.at[iv.at[c, pl.ds(cg * cgs, cgs)]],
                                  rows.at[c], sem).wait()

    def compute(cg, rows, acc):
        def grp(g, _):
            e0 = g * 16
            for j in range(16):
                e = e0 + j
                ev = _splat_i32(cg * cgs + e)
                av = [None] * nq
                for c in range(8):
                    ws = plsc.load_gather(wv, [_splat_i32(c), ev])
                    for qq in range(nq):
                        term = ws * rows[c, e, pl.ds(qq * 16, 16)]
                        av[qq] = term if c == 0 else av[qq] + term
                for qq in range(nq):
                    acc[e, pl.ds(qq * 16, 16)] = av[qq]
            return _
        lax.fori_loop(0, cgs // 16, grp, None)

    def scatter_fire(cg, acc, sem):
        pltpu.async_copy(acc, out_sh.at[dstv.at[cg]], sem, add=True)

    def scatter_wait(acc, sem):
        pltpu.make_async_copy(acc, out_sh.at[dstv.at[0]], sem).wait()

    pltpu.sync_copy(cnt_h.at[wid], cntv.at[pl.ds(0, 8)])
    cnt = cntv[pl.ds(0, 16)][0]
    nst = (cnt + (SBC - 1)) // SBC

    def stage(s, _):
        base = wid * EPSR + s * SBC
        sb0 = s * SBC
        pltpu.sync_copy(w8.at[:, pl.ds(base, SBC)], wv)
        pltpu.sync_copy(i8.at[:, pl.ds(base, SBC)], iv)
        pltpu.sync_copy(dst_h.at[pl.ds(base // cgs, SBC // cgs)], dstv)
        fire(0, rows_a, sem_a)

        def pair(p, _):
            ca = 2 * p
            cb = 2 * p + 1
            act_a = sb0 + ca * cgs < cnt
            act_b = sb0 + cb * cgs < cnt

            @pl.when(act_b)
            def _():
                fire(cb, rows_b, sem_b)

            @pl.when(jnp.logical_and(act_a, p > 0))
            def _():
                scatter_wait(acc_a, sem_sa)

            @pl.when(act_a)
            def _():
                drain(ca, rows_a, sem_a)
                compute(ca, rows_a, acc_a)
                scatter_fire(ca, acc_a, sem_sa)

            @pl.when(jnp.logical_and(p < SBC // (2 * cgs) - 1, sb0 + (ca + 2) * cgs < cnt))
            def _():
                fire(ca + 2, rows_a, sem_a)

            @pl.when(jnp.logical_and(act_b, p > 0))
            def _():
                scatter_wait(acc_b, sem_sb)

            @pl.when(act_b)
            def _():
                drain(cb, rows_b, sem_b)
                compute(cb, rows_b, acc_b)
                scatter_fire(cb, acc_b, sem_sb)
            return _

        lax.fori_loop(0, SBC // (2 * cgs), pair, None)
        scatter_wait(acc_a, sem_sa)

        @pl.when(sb0 + cgs < cnt)
        def _():
            scatter_wait(acc_b, sem_sb)
        return _

    lax.fori_loop(0, nst, stage, None)
    plsc.subcore_barrier()
    for j in range(5):
        r0 = sid * RPS + j * 128
        pltpu.sync_copy(out_sh.at[pl.ds(r0, 128)], out.at[scid, pl.ds(r0, 128)])


def _conv(y, w8, i8, dst_h, cnt_h, cout, cgs):
    body = functools.partial(_conv_body, cout, cgs)
    return pl.kernel(
        body,
        out_type=jax.ShapeDtypeStruct((2, NPAD, cout), jnp.float32),
        scratch_types=[
            pltpu.VMEM((8, SBC), jnp.float32),
            pltpu.VMEM((8, SBC), jnp.int32),
            pltpu.VMEM((SBC // cgs, cgs), jnp.int32),
            pltpu.VMEM((16,), jnp.int32),
            pltpu.VMEM((8, cgs, cout), jnp.float32),
            pltpu.VMEM((8, cgs, cout), jnp.float32),
            pltpu.VMEM((cgs, cout), jnp.float32),
            pltpu.VMEM((cgs, cout), jnp.float32),
            pltpu.VMEM((64, cout), jnp.float32),
            pltpu.VMEM_SHARED((NPAD, cout), jnp.float32),
            pltpu.SemaphoreType.DMA,
            pltpu.SemaphoreType.DMA,
            pltpu.SemaphoreType.DMA,
            pltpu.SemaphoreType.DMA,
        ],
        **_MESH,
    )(y, w8, i8, dst_h, cnt_h)


def _mm(x, w, b=None, relu=False):
    m, kin = x.shape
    kout = w.shape[1]
    bn = 1000
    bk = min(kout, 512)

    def body(*refs):
        x_ref, w_ref = refs[0], refs[1]
        o_ref = refs[-1]
        xx = x_ref[...]
        if relu:
            xx = jnp.maximum(xx, 0.0)
        o = jnp.dot(xx, w_ref[...], preferred_element_type=jnp.float32)
        if b is not None:
            o = o + refs[2][...]
        o_ref[...] = o

    in_specs = [pl.BlockSpec((bn, kin), lambda i, j: (i, 0)),
                pl.BlockSpec((kin, bk), lambda i, j: (0, j))]
    args = [x, w]
    if b is not None:
        in_specs.append(pl.BlockSpec((1, bk), lambda i, j: (0, j)))
        args.append(b.reshape(1, kout))
    return pl.pallas_call(
        body,
        grid=(m // bn, kout // bk),
        in_specs=in_specs,
        out_specs=pl.BlockSpec((bn, bk), lambda i, j: (i, j)),
        out_shape=jax.ShapeDtypeStruct((m, kout), jnp.float32),
    )(*args)


def _addk(arrays, scale=1.0):
    m, c = arrays[0].shape
    bn = 1000

    def body(*refs):
        o_ref = refs[-1]
        s = refs[0][...]
        for r in refs[1:-1]:
            s = s + r[...]
        o_ref[...] = s * scale

    return pl.pallas_call(
        body,
        grid=(m // bn,),
        in_specs=[pl.BlockSpec((bn, c), lambda i: (i, 0)) for _ in arrays],
        out_specs=pl.BlockSpec((bn, c), lambda i: (i, 0)),
        out_shape=jax.ShapeDtypeStruct((m, c), jnp.float32),
    )(*arrays)


def _kflat(kk, cout_pad=None):
    cin, cout = kk.shape[3], kk.shape[4]
    k = kk.reshape(64, cin, cout)
    if cout_pad is not None and cout_pad != cout:
        k = jnp.pad(k, ((0, 0), (0, 0), (0, cout_pad - cout)))
        cout = cout_pad
    return k.transpose(1, 0, 2).reshape(cin, 64 * cout)


def kernel(pos, vel, box, box_feats, edge_index_fluid, edge_index_obstacle,
           k0f, k0o, d0W, d0b, k1, d1W, d1b, k2, d2W, d2b, k3, d3W, d3b):
    px, py, pz = pos[:, 0], pos[:, 1], pos[:, 2]
    bx, by, bz = box[:, 0], box[:, 1], box[:, 2]
    ep = EPAD - E
    srcf = jnp.pad(edge_index_fluid[0], (0, ep))
    dstf = jnp.pad(edge_index_fluid[1], (0, ep))
    srco = jnp.pad(edge_index_obstacle[0], (0, ep))
    dsto = jnp.pad(edge_index_obstacle[1], (0, ep))
    wf8, if8, dcf, ccf = _geom(px, py, pz, px, py, pz, srcf, dstf)
    wo8, io8, dco, cco = _geom(bx, by, bz, px, py, pz, srco, dsto)

    fluid_feats = jnp.concatenate([jnp.ones_like(pos[:, 0:1]), vel], axis=-1)
    ff_pad = jnp.pad(fluid_feats, ((0, 0), (0, 4)))
    bf_pad = jnp.pad(box_feats, ((0, 0), (0, 5)))

    def conv(x_pad, kflat_mat, w8, i8, dst_h, cnt_h, cout, relu):
        cgs = 64
        p = _conv(_mm(x_pad, kflat_mat, relu=relu).reshape(N * 64, cout),
                  w8, i8, dst_h.reshape(NSUB * EPSR // cgs, cgs), cnt_h,
                  cout, cgs)
        return p[0, :N], p[1, :N]

    k0f_f = _kflat(jnp.pad(k0f, ((0, 0),) * 3 + ((0, 4), (0, 0))))
    k0o_f = _kflat(jnp.pad(k0o, ((0, 0),) * 3 + ((0, 5), (0, 0))))
    c0f = _addk(conv(ff_pad, k0f_f, wf8, if8, dcf, ccf, 32, False))
    c0o = _addk(conv(bf_pad, k0o_f, wo8, io8, dco, cco, 32, False))
    d0 = _mm(ff_pad, jnp.pad(d0W, ((0, 4), (0, 0))), d0b, False)
    feats = jnp.concatenate([c0o, c0f, d0], axis=-1)

    p0, p1 = conv(feats, _kflat(k1), wf8, if8, dcf, ccf, 64, True)
    d1 = _mm(feats, d1W, d1b, True)
    ans1 = _addk((p0, p1, d1))
    p0, p1 = conv(ans1, _kflat(k2), wf8, if8, dcf, ccf, 64, True)
    d2 = _mm(ans1, d2W, d2b, True)
    ans2 = _addk((p0, p1, d2, ans1))
    p0, p1 = conv(ans2, _kflat(k3, cout_pad=16), wf8, if8, dcf, ccf, 16, True)
    d3 = _mm(ans2, jnp.pad(d3W, ((0, 0), (0, 13))),
             jnp.pad(d3b, (0, 13)), True)
    ans3 = _addk((p0, p1, d3), scale=1.0 / 128.0)
    return ans3[:, :3]


# speedup vs baseline: 1.1937x; 1.0544x over previous
import functools

import jax
import jax.numpy as jnp
from jax import lax
from jax.experimental import pallas as pl
from jax.experimental.pallas import tpu as pltpu
from jax.experimental.pallas import tpu_sc as plsc
import numpy as np

N = 10000
E = 160000
EXTENT = np.float32(1.5 * 6 * 0.025)
NSUB = 32
SB = 1024
NSTG = 5
EPS = SB * NSTG
EPAD = NSUB * EPS
CG = 64
SBC = 512
NPAIR = SBC // (2 * CG)
EPSR = EPS + SB
NSTGR = EPSR // SBC
NPAD = 10240
RPS = NPAD // 16

_MESH = dict(mesh=plsc.VectorSubcoreMesh(core_axis_name="c", subcore_axis_name="s"),
             compiler_params=pltpu.CompilerParams(needs_layout_passes=False,
                                                  use_tc_tiling_on_sc=False))

def _wid():
    return lax.axis_index("s") * 2 + lax.axis_index("c")


def _splat_i32(v):
    return (jnp.broadcast_to(jnp.int32(v), (16,)) if isinstance(v, int)
            else lax.broadcast(v, (16,)))


def _sqrt16(x):
    i = plsc.bitcast(x, jnp.int32)
    y = plsc.bitcast(lax.shift_right_logical(i, 1) + jnp.int32(0x1FBD1DF5), jnp.float32)
    for _ in range(3):
        y = 0.5 * (y + x / y)
    return y


def _geom_body(sx_h, sy_h, sz_h, dx_h, dy_h, dz_h, src_h, dst_h,
               w_out, i_out, d_out, c_out,
               sx, sy, sz, dx, dy, dz, srcv, dstv, wbuf, ibuf, dbuf, cntbuf):
    wid = _wid()
    pltpu.sync_copy(sx_h, sx)
    pltpu.sync_copy(sy_h, sy)
    pltpu.sync_copy(sz_h, sz)
    pltpu.sync_copy(dx_h, dx)
    pltpu.sync_copy(dy_h, dy)
    pltpu.sync_copy(dz_h, dz)
    scale = jnp.float32(2.0 / EXTENT)
    zf = jnp.zeros((16,), jnp.float32)
    zi = jnp.zeros((16,), jnp.int32)
    cnt = jnp.int32(0)

    for kc in range(NSTG):
        base = wid * EPS + kc * SB
        pltpu.sync_copy(src_h.at[pl.ds(base, SB)], srcv)
        pltpu.sync_copy(dst_h.at[pl.ds(base, SB)], dstv)

        def zrow(r, _):
            for c in range(8):
                wbuf[c, pl.ds(r * 16, 16)] = zf
                ibuf[c, pl.ds(r * 16, 16)] = zi
            dbuf[pl.ds(r * 16, 16)] = zi
            return _
        lax.fori_loop(0, (SB + 16) // 16, zrow, None)

        def grp(g, off):
            s16 = srcv[pl.ds(g * 16, 16)]
            d16 = dstv[pl.ds(g * 16, 16)]
            xs = plsc.load_gather(sx, [s16])
            ys = plsc.load_gather(sy, [s16])
            zs = plsc.load_gather(sz, [s16])
            xd = plsc.load_gather(dx, [d16])
            yd = plsc.load_gather(dy, [d16])
            zd = plsc.load_gather(dz, [d16])
            rx = (xs - xd) * scale
            ry = (ys - yd) * scale
            rz = (zs - zd) * scale
            r2 = rx * rx + ry * ry + rz * rz
            t = 1.0 - r2
            win = jnp.clip(t * t * t, 0.0, 1.0)
            eid = base + g * 16 + lax.iota(jnp.int32, 16)
            win = jnp.where(eid < E, win, 0.0)
            mask = win > 0.0
            l2 = _sqrt16(jnp.maximum(r2, 1e-12))
            linf = jnp.maximum(
                jnp.maximum(jnp.maximum(jnp.abs(rx), jnp.abs(ry)), jnp.abs(rz)),
                1e-12)
            q = l2 / linf
            ax = jnp.clip(rx * q, -1.0, 1.0)
            ay = jnp.clip(ry * q, -1.0, 1.0)
            az = jnp.clip(rz * q, -1.0, 1.0)
            ux = (ax + 1.0) * 1.5
            uy = (ay + 1.0) * 1.5
            uz = (az + 1.0) * 1.5
            lx = jnp.minimum(ux.astype(jnp.int32), 2)
            ly = jnp.minimum(uy.astype(jnp.int32), 2)
            lz = jnp.minimum(uz.astype(jnp.int32), 2)
            fx = ux - lx.astype(jnp.float32)
            fy = uy - ly.astype(jnp.float32)
            fz = uz - lz.astype(jnp.float32)
            base_i = s16 * 64 + lx * 16 + ly * 4 + lz
            for c in range(8):
                ox, oy, oz = (c >> 2) & 1, (c >> 1) & 1, c & 1
                w = ((fx if ox else 1.0 - fx)
                     * (fy if oy else 1.0 - fy)
                     * (fz if oz else 1.0 - fz) * win)
                gi = base_i + (ox * 16 + oy * 4 + oz)
                plsc.store_compressed(wbuf.at[c, pl.ds(off, 16)], w, mask=mask)
                plsc.store_compressed(ibuf.at[c, pl.ds(off, 16)], gi, mask=mask)
            plsc.store_compressed(dbuf.at[pl.ds(off, 16)], d16, mask=mask)
            return off + jnp.sum(jnp.where(mask, 1, 0).astype(jnp.int32))

        off = lax.fori_loop(0, SB // 16, grp, jnp.int32(0))
        cnt = pl.multiple_of(cnt, 8)
        pltpu.sync_copy(wbuf.at[:, pl.ds(0, SB)],
                        w_out.at[:, pl.ds(wid * EPSR + cnt, SB)])
        pltpu.sync_copy(ibuf.at[:, pl.ds(0, SB)],
                        i_out.at[:, pl.ds(wid * EPSR + cnt, SB)])
        pltpu.sync_copy(dbuf.at[pl.ds(0, SB)],
                        d_out.at[pl.ds(wid * EPSR + cnt, SB)])
        cnt = cnt + jnp.bitwise_and(off + 7, jnp.int32(-8))

    cnt = pl.multiple_of(cnt, 8)
    def zrow2(r, _):
        for c in range(8):
            wbuf[c, pl.ds(r * 16, 16)] = zf
            ibuf[c, pl.ds(r * 16, 16)] = zi
        dbuf[pl.ds(r * 16, 16)] = zi
        return _
    lax.fori_loop(0, (SB + 16) // 16, zrow2, None)
    pltpu.sync_copy(wbuf.at[:, pl.ds(0, SB)],
                    w_out.at[:, pl.ds(wid * EPSR + cnt, SB)])
    pltpu.sync_copy(ibuf.at[:, pl.ds(0, SB)],
                    i_out.at[:, pl.ds(wid * EPSR + cnt, SB)])
    pltpu.sync_copy(dbuf.at[pl.ds(0, SB)],
                    d_out.at[pl.ds(wid * EPSR + cnt, SB)])

    cntbuf[pl.ds(0, 16)] = lax.broadcast(cnt, (16,))
    pltpu.sync_copy(cntbuf.at[pl.ds(0, 8)], c_out.at[wid])


_geom = functools.partial(
    pl.kernel,
    out_type=[jax.ShapeDtypeStruct((8, NSUB * EPSR), jnp.float32),
              jax.ShapeDtypeStruct((8, NSUB * EPSR), jnp.int32),
              jax.ShapeDtypeStruct((NSUB * EPSR,), jnp.int32),
              jax.ShapeDtypeStruct((NSUB, 8), jnp.int32)],
    scratch_types=[pltpu.VMEM((N,), jnp.float32)] * 6
    + [pltpu.VMEM((SB,), jnp.int32)] * 2
    + [pltpu.VMEM((8, SB + 16), jnp.float32), pltpu.VMEM((8, SB + 16), jnp.int32),
       pltpu.VMEM((SB + 16,), jnp.int32), pltpu.VMEM((16,), jnp.int32)],
    **_MESH,
)(_geom_body)


def _conv_body(cout, cgs, y, w8, i8, dst_h, cnt_h, out,
               wv, iv, dstv, cntv, rows_a, rows_b, acc_a, acc_b, zbuf, out_sh,
               sem_a, sem_b, sem_sa, sem_sb):
    wid = _wid()
    scid = lax.axis_index("c")
    sid = lax.axis_index("s")
    nq = cout // 16

    def zrow(r, _):
        for qq in range(nq):
            zbuf[r, pl.ds(qq * 16, 16)] = jnp.zeros((16,), jnp.float32)
        return _
    lax.fori_loop(0, 64, zrow, None)
    for j in range(10):
        pltpu.sync_copy(zbuf, out_sh.at[pl.ds(sid * RPS + j * 64, 64)])
    plsc.subcore_barrier()

    def fire(cg, rows, sem):
        for c in range(8):
            pltpu.async_copy(y.at[iv.at[c, pl.ds(cg * cgs, cgs)]], rows.at[c], sem)

    def drain(cg, rows, sem):
        for c in range(8):
            pltpu.make_async_copy(y.at[iv.at[c, pl.ds(cg * cgs, cgs)]],
                                  rows.at[c], sem).wait()

    def compute(cg, rows, acc):
        def grp(g, _):
            e0 = g * 16
            for j in range(16):
                e = e0 + j
                ev = _splat_i32(cg * cgs + e)
                av = [None] * nq
                for c in range(8):
                    ws = plsc.load_gather(wv, [_splat_i32(c), ev])
                    for qq in range(nq):
                        term = ws * rows[c, e, pl.ds(qq * 16, 16)]
                        av[qq] = term if c == 0 else av[qq] + term
                for qq in range(nq):
                    acc[e, pl.ds(qq * 16, 16)] = av[qq]
            return _
        lax.fori_loop(0, cgs // 16, grp, None)

    def scatter_fire(cg, acc, sem):
        pltpu.async_copy(acc, out_sh.at[dstv.at[cg]], sem, add=True)

    def scatter_wait(acc, sem):
        pltpu.make_async_copy(acc, out_sh.at[dstv.at[0]], sem).wait()

    pltpu.sync_copy(cnt_h.at[wid], cntv.at[pl.ds(0, 8)])
    cnt = cntv[pl.ds(0, 16)][0]
    nst = (cnt + (SBC - 1)) // SBC

    def stage(s, _):
        base = wid * EPSR + s * SBC
        sb0 = s * SBC
        pltpu.sync_copy(w8.at[:, pl.ds(base, SBC)], wv)
        pltpu.sync_copy(i8.at[:, pl.ds(base, SBC)], iv)
        pltpu.sync_copy(dst_h.at[pl.ds(base // cgs, SBC // cgs)], dstv)
        fire(0, rows_a, sem_a)

        def pair(p, _):
            ca = 2 * p
            cb = 2 * p + 1
            act_a = sb0 + ca * cgs < cnt
            act_b = sb0 + cb * cgs < cnt

            @pl.when(act_b)
            def _():
                fire(cb, rows_b, sem_b)

            @pl.when(jnp.logical_and(act_a, p > 0))
            def _():
                scatter_wait(acc_a, sem_sa)

            @pl.when(act_a)
            def _():
                drain(ca, rows_a, sem_a)
                compute(ca, rows_a, acc_a)
                scatter_fire(ca, acc_a, sem_sa)

            @pl.when(jnp.logical_and(p < SBC // (2 * cgs) - 1, sb0 + (ca + 2) * cgs < cnt))
            def _():
                fire(ca + 2, rows_a, sem_a)

            @pl.when(jnp.logical_and(act_b, p > 0))
            def _():
                scatter_wait(acc_b, sem_sb)

            @pl.when(act_b)
            def _():
                drain(cb, rows_b, sem_b)
                compute(cb, rows_b, acc_b)
                scatter_fire(cb, acc_b, sem_sb)
            return _

        lax.fori_loop(0, SBC // (2 * cgs), pair, None)
        scatter_wait(acc_a, sem_sa)

        @pl.when(sb0 + cgs < cnt)
        def _():
            scatter_wait(acc_b, sem_sb)
        return _

    lax.fori_loop(0, nst, stage, None)
    plsc.subcore_barrier()
    for j in range(5):
        r0 = sid * RPS + j * 128
        pltpu.sync_copy(out_sh.at[pl.ds(r0, 128)], out.at[scid, pl.ds(r0, 128)])


def _conv(y, w8, i8, dst_h, cnt_h, cout, cgs):
    body = functools.partial(_conv_body, cout, cgs)
    return pl.kernel(
        body,
        out_type=jax.ShapeDtypeStruct((2, NPAD, cout), jnp.float32),
        scratch_types=[
            pltpu.VMEM((8, SBC), jnp.float32),
            pltpu.VMEM((8, SBC), jnp.int32),
            pltpu.VMEM((SBC // cgs, cgs), jnp.int32),
            pltpu.VMEM((16,), jnp.int32),
            pltpu.VMEM((8, cgs, cout), jnp.float32),
            pltpu.VMEM((8, cgs, cout), jnp.float32),
            pltpu.VMEM((cgs, cout), jnp.float32),
            pltpu.VMEM((cgs, cout), jnp.float32),
            pltpu.VMEM((64, cout), jnp.float32),
            pltpu.VMEM_SHARED((NPAD, cout), jnp.float32),
            pltpu.SemaphoreType.DMA,
            pltpu.SemaphoreType.DMA,
            pltpu.SemaphoreType.DMA,
            pltpu.SemaphoreType.DMA,
        ],
        **_MESH,
    )(y, w8, i8, dst_h, cnt_h)


def _mm(x, w, b=None, relu=False):
    m, kin = x.shape
    kout = w.shape[1]
    bn = 1000
    bk = min(kout, 512)

    def body(*refs):
        x_ref, w_ref = refs[0], refs[1]
        o_ref = refs[-1]
        xx = x_ref[...]
        if relu:
            xx = jnp.maximum(xx, 0.0)
        o = jnp.dot(xx, w_ref[...], preferred_element_type=jnp.float32)
        if b is not None:
            o = o + refs[2][...]
        o_ref[...] = o

    in_specs = [pl.BlockSpec((bn, kin), lambda i, j: (i, 0)),
                pl.BlockSpec((kin, bk), lambda i, j: (0, j))]
    args = [x, w]
    if b is not None:
        in_specs.append(pl.BlockSpec((1, bk), lambda i, j: (0, j)))
        args.append(b.reshape(1, kout))
    return pl.pallas_call(
        body,
        grid=(m // bn, kout // bk),
        in_specs=in_specs,
        out_specs=pl.BlockSpec((bn, bk), lambda i, j: (i, j)),
        out_shape=jax.ShapeDtypeStruct((m, kout), jnp.float32),
    )(*args)


def _addk(arrays, scale=1.0):
    m, c = arrays[0].shape
    bn = 1000

    def body(*refs):
        o_ref = refs[-1]
        s = refs[0][...]
        for r in refs[1:-1]:
            s = s + r[...]
        o_ref[...] = s * scale

    return pl.pallas_call(
        body,
        grid=(m // bn,),
        in_specs=[pl.BlockSpec((bn, c), lambda i: (i, 0)) for _ in arrays],
        out_specs=pl.BlockSpec((bn, c), lambda i: (i, 0)),
        out_shape=jax.ShapeDtypeStruct((m, c), jnp.float32),
    )(*arrays)


def _kflat(kk, cout_pad=None):
    cin, cout = kk.shape[3], kk.shape[4]
    k = kk.reshape(64, cin, cout)
    if cout_pad is not None and cout_pad != cout:
        k = jnp.pad(k, ((0, 0), (0, 0), (0, cout_pad - cout)))
        cout = cout_pad
    return k.transpose(1, 0, 2).reshape(cin, 64 * cout)


def kernel(pos, vel, box, box_feats, edge_index_fluid, edge_index_obstacle,
           k0f, k0o, d0W, d0b, k1, d1W, d1b, k2, d2W, d2b, k3, d3W, d3b):
    px, py, pz = pos[:, 0], pos[:, 1], pos[:, 2]
    bx, by, bz = box[:, 0], box[:, 1], box[:, 2]
    ep = EPAD - E
    srcf = jnp.pad(edge_index_fluid[0], (0, ep))
    dstf = jnp.pad(edge_index_fluid[1], (0, ep))
    srco = jnp.pad(edge_index_obstacle[0], (0, ep))
    dsto = jnp.pad(edge_index_obstacle[1], (0, ep))
    wf8, if8, dcf, ccf = _geom(px, py, pz, px, py, pz, srcf, dstf)
    wo8, io8, dco, cco = _geom(bx, by, bz, px, py, pz, srco, dsto)

    fluid_feats = jnp.concatenate([jnp.ones_like(pos[:, 0:1]), vel], axis=-1)
    ff_pad = jnp.pad(fluid_feats, ((0, 0), (0, 4)))
    bf_pad = jnp.pad(box_feats, ((0, 0), (0, 5)))

    def conv(x_pad, kflat_mat, w8, i8, dst_h, cnt_h, cout, relu):
        cgs = 32 if cout == 64 else 64
        p = _conv(_mm(x_pad, kflat_mat, relu=relu).reshape(N * 64, cout),
                  w8, i8, dst_h.reshape(NSUB * EPSR // cgs, cgs), cnt_h,
                  cout, cgs)
        return p[0, :N], p[1, :N]

    k0f_f = _kflat(jnp.pad(k0f, ((0, 0),) * 3 + ((0, 4), (0, 0))))
    k0o_f = _kflat(jnp.pad(k0o, ((0, 0),) * 3 + ((0, 5), (0, 0))))
    c0f = _addk(conv(ff_pad, k0f_f, wf8, if8, dcf, ccf, 32, False))
    c0o = _addk(conv(bf_pad, k0o_f, wo8, io8, dco, cco, 32, False))
    d0 = _mm(ff_pad, jnp.pad(d0W, ((0, 4), (0, 0))), d0b, False)
    feats = jnp.concatenate([c0o, c0f, d0], axis=-1)

    p0, p1 = conv(feats, _kflat(k1), wf8, if8, dcf, ccf, 64, True)
    d1 = _mm(feats, d1W, d1b, True)
    ans1 = _addk((p0, p1, d1))
    p0, p1 = conv(ans1, _kflat(k2), wf8, if8, dcf, ccf, 64, True)
    d2 = _mm(ans1, d2W, d2b, True)
    ans2 = _addk((p0, p1, d2, ans1))
    p0, p1 = conv(ans2, _kflat(k3, cout_pad=16), wf8, if8, dcf, ccf, 16, True)
    d3 = _mm(ans2, jnp.pad(d3W, ((0, 0), (0, 13))),
             jnp.pad(d3b, (0, 13)), True)
    ans3 = _addk((p0, p1, d3), scale=1.0 / 128.0)
    return ans3[:, :3]


# speedup vs baseline: 1.2340x; 1.0337x over previous
import functools

import jax
import jax.numpy as jnp
from jax import lax
from jax.experimental import pallas as pl
from jax.experimental.pallas import tpu as pltpu
from jax.experimental.pallas import tpu_sc as plsc
import numpy as np

N = 10000
E = 160000
EXTENT = np.float32(1.5 * 6 * 0.025)
NSUB = 32
SB = 1024
NSTG = 5
EPS = SB * NSTG
EPAD = NSUB * EPS
CG = 64
SBC = 512
NPAIR = SBC // (2 * CG)
EPSR = EPS + SB
NSTGR = EPSR // SBC
NPAD = 10240
RPS = NPAD // 16

_MESH = dict(mesh=plsc.VectorSubcoreMesh(core_axis_name="c", subcore_axis_name="s"),
             compiler_params=pltpu.CompilerParams(needs_layout_passes=False,
                                                  use_tc_tiling_on_sc=False))

def _wid():
    return lax.axis_index("s") * 2 + lax.axis_index("c")


def _splat_i32(v):
    return (jnp.broadcast_to(jnp.int32(v), (16,)) if isinstance(v, int)
            else lax.broadcast(v, (16,)))


def _sqrt16(x):
    i = plsc.bitcast(x, jnp.int32)
    y = plsc.bitcast(lax.shift_right_logical(i, 1) + jnp.int32(0x1FBD1DF5), jnp.float32)
    for _ in range(3):
        y = 0.5 * (y + x / y)
    return y


def _geom_body(sx_h, sy_h, sz_h, dx_h, dy_h, dz_h, src_h, dst_h,
               w_out, i_out, d_out, c_out,
               sx, sy, sz, dx, dy, dz, srcv, dstv, wbuf, ibuf, dbuf, cntbuf):
    wid = _wid()
    pltpu.sync_copy(sx_h, sx)
    pltpu.sync_copy(sy_h, sy)
    pltpu.sync_copy(sz_h, sz)
    pltpu.sync_copy(dx_h, dx)
    pltpu.sync_copy(dy_h, dy)
    pltpu.sync_copy(dz_h, dz)
    scale = jnp.float32(2.0 / EXTENT)
    zf = jnp.zeros((16,), jnp.float32)
    zi = jnp.zeros((16,), jnp.int32)
    cnt = jnp.int32(0)

    for kc in range(NSTG):
        base = wid * EPS + kc * SB
        pltpu.sync_copy(src_h.at[pl.ds(base, SB)], srcv)
        pltpu.sync_copy(dst_h.at[pl.ds(base, SB)], dstv)

        def zrow(r, _):
            for c in range(8):
                wbuf[c, pl.ds(r * 16, 16)] = zf
                ibuf[c, pl.ds(r * 16, 16)] = zi
            dbuf[pl.ds(r * 16, 16)] = zi
            return _
        lax.fori_loop(0, (SB + 16) // 16, zrow, None)

        def grp(g, off):
            s16 = srcv[pl.ds(g * 16, 16)]
            d16 = dstv[pl.ds(g * 16, 16)]
            xs = plsc.load_gather(sx, [s16])
            ys = plsc.load_gather(sy, [s16])
            zs = plsc.load_gather(sz, [s16])
            xd = plsc.load_gather(dx, [d16])
            yd = plsc.load_gather(dy, [d16])
            zd = plsc.load_gather(dz, [d16])
            rx = (xs - xd) * scale
            ry = (ys - yd) * scale
            rz = (zs - zd) * scale
            r2 = rx * rx + ry * ry + rz * rz
            t = 1.0 - r2
            win = jnp.clip(t * t * t, 0.0, 1.0)
            eid = base + g * 16 + lax.iota(jnp.int32, 16)
            win = jnp.where(eid < E, win, 0.0)
            mask = win > 0.0
            l2 = _sqrt16(jnp.maximum(r2, 1e-12))
            linf = jnp.maximum(
                jnp.maximum(jnp.maximum(jnp.abs(rx), jnp.abs(ry)), jnp.abs(rz)),
                1e-12)
            q = l2 / linf
            ax = jnp.clip(rx * q, -1.0, 1.0)
            ay = jnp.clip(ry * q, -1.0, 1.0)
            az = jnp.clip(rz * q, -1.0, 1.0)
            ux = (ax + 1.0) * 1.5
            uy = (ay + 1.0) * 1.5
            uz = (az + 1.0) * 1.5
            lx = jnp.minimum(ux.astype(jnp.int32), 2)
            ly = jnp.minimum(uy.astype(jnp.int32), 2)
            lz = jnp.minimum(uz.astype(jnp.int32), 2)
            fx = ux - lx.astype(jnp.float32)
            fy = uy - ly.astype(jnp.float32)
            fz = uz - lz.astype(jnp.float32)
            base_i = s16 * 64 + lx * 16 + ly * 4 + lz
            for c in range(8):
                ox, oy, oz = (c >> 2) & 1, (c >> 1) & 1, c & 1
                w = ((fx if ox else 1.0 - fx)
                     * (fy if oy else 1.0 - fy)
                     * (fz if oz else 1.0 - fz) * win)
                gi = base_i + (ox * 16 + oy * 4 + oz)
                plsc.store_compressed(wbuf.at[c, pl.ds(off, 16)], w, mask=mask)
                plsc.store_compressed(ibuf.at[c, pl.ds(off, 16)], gi, mask=mask)
            plsc.store_compressed(dbuf.at[pl.ds(off, 16)], d16, mask=mask)
            return off + jnp.sum(jnp.where(mask, 1, 0).astype(jnp.int32))

        off = lax.fori_loop(0, SB // 16, grp, jnp.int32(0))
        cnt = pl.multiple_of(cnt, 8)
        pltpu.sync_copy(wbuf.at[:, pl.ds(0, SB)],
                        w_out.at[:, pl.ds(wid * EPSR + cnt, SB)])
        pltpu.sync_copy(ibuf.at[:, pl.ds(0, SB)],
                        i_out.at[:, pl.ds(wid * EPSR + cnt, SB)])
        pltpu.sync_copy(dbuf.at[pl.ds(0, SB)],
                        d_out.at[pl.ds(wid * EPSR + cnt, SB)])
        cnt = cnt + jnp.bitwise_and(off + 7, jnp.int32(-8))

    cnt = pl.multiple_of(cnt, 8)
    def zrow2(r, _):
        for c in range(8):
            wbuf[c, pl.ds(r * 16, 16)] = zf
            ibuf[c, pl.ds(r * 16, 16)] = zi
        dbuf[pl.ds(r * 16, 16)] = zi
        return _
    lax.fori_loop(0, (SB + 16) // 16, zrow2, None)
    pltpu.sync_copy(wbuf.at[:, pl.ds(0, SB)],
                    w_out.at[:, pl.ds(wid * EPSR + cnt, SB)])
    pltpu.sync_copy(ibuf.at[:, pl.ds(0, SB)],
                    i_out.at[:, pl.ds(wid * EPSR + cnt, SB)])
    pltpu.sync_copy(dbuf.at[pl.ds(0, SB)],
                    d_out.at[pl.ds(wid * EPSR + cnt, SB)])

    cntbuf[pl.ds(0, 16)] = lax.broadcast(cnt, (16,))
    pltpu.sync_copy(cntbuf.at[pl.ds(0, 8)], c_out.at[wid])


_geom = functools.partial(
    pl.kernel,
    out_type=[jax.ShapeDtypeStruct((8, NSUB * EPSR), jnp.float32),
              jax.ShapeDtypeStruct((8, NSUB * EPSR), jnp.int32),
              jax.ShapeDtypeStruct((NSUB * EPSR,), jnp.int32),
              jax.ShapeDtypeStruct((NSUB, 8), jnp.int32)],
    scratch_types=[pltpu.VMEM((N,), jnp.float32)] * 6
    + [pltpu.VMEM((SB,), jnp.int32)] * 2
    + [pltpu.VMEM((8, SB + 16), jnp.float32), pltpu.VMEM((8, SB + 16), jnp.int32),
       pltpu.VMEM((SB + 16,), jnp.int32), pltpu.VMEM((16,), jnp.int32)],
    **_MESH,
)(_geom_body)


def _conv_body(cout, cgs, y, w8, i8, dst_h, cnt_h, out,
               wv, iv, dstv, cntv, rows_a, rows_b, acc_a, acc_b, zbuf, out_sh,
               sem_a, sem_b, sem_sa, sem_sb):
    wid = _wid()
    scid = lax.axis_index("c")
    sid = lax.axis_index("s")
    nq = cout // 16

    def zrow(r, _):
        for qq in range(nq):
            zbuf[r, pl.ds(qq * 16, 16)] = jnp.zeros((16,), jnp.float32)
        return _
    lax.fori_loop(0, 64, zrow, None)
    for j in range(10):
        pltpu.sync_copy(zbuf, out_sh.at[pl.ds(sid * RPS + j * 64, 64)])
    plsc.subcore_barrier()

    def fire(cg, rows, sem):
        for c in range(8):
            pltpu.async_copy(y.at[iv.at[c, pl.ds(cg * cgs, cgs)]], rows.at[c], sem)

    def drain(cg, rows, sem):
        for c in range(8):
            pltpu.make_async_copy(y.at[iv.at[c, pl.ds(cg * cgs, cgs)]],
                                  rows.at[c], sem).wait()

    def compute(cg, rows, acc):
        def grp(g, _):
            e0 = g * 16
            for j in range(16):
                e = e0 + j
                ev = _splat_i32(cg * cgs + e)
                av = [None] * nq
                for c in range(8):
                    ws = plsc.load_gather(wv, [_splat_i32(c), ev])
                    for qq in range(nq):
                        term = ws * rows[c, e, pl.ds(qq * 16, 16)]
                        av[qq] = term if c == 0 else av[qq] + term
                for qq in range(nq):
                    acc[e, pl.ds(qq * 16, 16)] = av[qq]
            return _
        lax.fori_loop(0, cgs // 16, grp, None)

    def scatter_fire(cg, acc, sem):
        pltpu.async_copy(acc, out_sh.at[dstv.at[cg]], sem, add=True)

    def scatter_wait(acc, sem):
        pltpu.make_async_copy(acc, out_sh.at[dstv.at[0]], sem).wait()

    pltpu.sync_copy(cnt_h.at[wid], cntv.at[pl.ds(0, 8)])
    cnt = cntv[pl.ds(0, 16)][0]
    nst = (cnt + (SBC - 1)) // SBC

    def stage(s, _):
        base = wid * EPSR + s * SBC
        sb0 = s * SBC
        pltpu.sync_copy(w8.at[:, pl.ds(base, SBC)], wv)
        pltpu.sync_copy(i8.at[:, pl.ds(base, SBC)], iv)
        pltpu.sync_copy(dst_h.at[pl.ds(base // cgs, SBC // cgs)], dstv)
        fire(0, rows_a, sem_a)

        def pair(p, _):
            ca = 2 * p
            cb = 2 * p + 1
            act_a = sb0 + ca * cgs < cnt
            act_b = sb0 + cb * cgs < cnt

            @pl.when(act_b)
            def _():
                fire(cb, rows_b, sem_b)

            @pl.when(jnp.logical_and(act_a, p > 0))
            def _():
                scatter_wait(acc_a, sem_sa)

            @pl.when(act_a)
            def _():
                drain(ca, rows_a, sem_a)
                compute(ca, rows_a, acc_a)
                scatter_fire(ca, acc_a, sem_sa)

            @pl.when(jnp.logical_and(p < SBC // (2 * cgs) - 1, sb0 + (ca + 2) * cgs < cnt))
            def _():
                fire(ca + 2, rows_a, sem_a)

            @pl.when(jnp.logical_and(act_b, p > 0))
            def _():
                scatter_wait(acc_b, sem_sb)

            @pl.when(act_b)
            def _():
                drain(cb, rows_b, sem_b)
                compute(cb, rows_b, acc_b)
                scatter_fire(cb, acc_b, sem_sb)
            return _

        lax.fori_loop(0, SBC // (2 * cgs), pair, None)
        scatter_wait(acc_a, sem_sa)

        @pl.when(sb0 + cgs < cnt)
        def _():
            scatter_wait(acc_b, sem_sb)
        return _

    lax.fori_loop(0, nst, stage, None)
    plsc.subcore_barrier()
    for j in range(5):
        r0 = sid * RPS + j * 128
        pltpu.sync_copy(out_sh.at[pl.ds(r0, 128)], out.at[scid, pl.ds(r0, 128)])


def _conv(y, w8, i8, dst_h, cnt_h, cout, cgs):
    body = functools.partial(_conv_body, cout, cgs)
    return pl.kernel(
        body,
        out_type=jax.ShapeDtypeStruct((2, NPAD, cout), jnp.float32),
        scratch_types=[
            pltpu.VMEM((8, SBC), jnp.float32),
            pltpu.VMEM((8, SBC), jnp.int32),
            pltpu.VMEM((SBC // cgs, cgs), jnp.int32),
            pltpu.VMEM((16,), jnp.int32),
            pltpu.VMEM((8, cgs, cout), jnp.float32),
            pltpu.VMEM((8, cgs, cout), jnp.float32),
            pltpu.VMEM((cgs, cout), jnp.float32),
            pltpu.VMEM((cgs, cout), jnp.float32),
            pltpu.VMEM((64, cout), jnp.float32),
            pltpu.VMEM_SHARED((NPAD, cout), jnp.float32),
            pltpu.SemaphoreType.DMA,
            pltpu.SemaphoreType.DMA,
            pltpu.SemaphoreType.DMA,
            pltpu.SemaphoreType.DMA,
        ],
        **_MESH,
    )(y, w8, i8, dst_h, cnt_h)


def _mm(x, w, b=None, relu=False):
    m, kin = x.shape
    kout = w.shape[1]
    bn = 1000
    bk = min(kout, 512)

    def body(*refs):
        x_ref, w_ref = refs[0], refs[1]
        o_ref = refs[-1]
        xx = x_ref[...]
        if relu:
            xx = jnp.maximum(xx, 0.0)
        o = jnp.dot(xx, w_ref[...], preferred_element_type=jnp.float32)
        if b is not None:
            o = o + refs[2][...]
        o_ref[...] = o

    in_specs = [pl.BlockSpec((bn, kin), lambda i, j: (i, 0)),
                pl.BlockSpec((kin, bk), lambda i, j: (0, j))]
    args = [x, w]
    if b is not None:
        in_specs.append(pl.BlockSpec((1, bk), lambda i, j: (0, j)))
        args.append(b.reshape(1, kout))
    return pl.pallas_call(
        body,
        grid=(m // bn, kout // bk),
        in_specs=in_specs,
        out_specs=pl.BlockSpec((bn, bk), lambda i, j: (i, j)),
        out_shape=jax.ShapeDtypeStruct((m, kout), jnp.float32),
    )(*args)


def _addk(arrays, scale=1.0):
    m, c = arrays[0].shape
    bn = 1000

    def body(*refs):
        o_ref = refs[-1]
        s = refs[0][...]
        for r in refs[1:-1]:
            s = s + r[...]
        o_ref[...] = s * scale

    return pl.pallas_call(
        body,
        grid=(m // bn,),
        in_specs=[pl.BlockSpec((bn, c), lambda i: (i, 0)) for _ in arrays],
        out_specs=pl.BlockSpec((bn, c), lambda i: (i, 0)),
        out_shape=jax.ShapeDtypeStruct((m, c), jnp.float32),
    )(*arrays)


def _kflat(kk, cout_pad=None):
    cin, cout = kk.shape[3], kk.shape[4]
    k = kk.reshape(64, cin, cout)
    if cout_pad is not None and cout_pad != cout:
        k = jnp.pad(k, ((0, 0), (0, 0), (0, cout_pad - cout)))
        cout = cout_pad
    return k.transpose(1, 0, 2).reshape(cin, 64 * cout)


def kernel(pos, vel, box, box_feats, edge_index_fluid, edge_index_obstacle,
           k0f, k0o, d0W, d0b, k1, d1W, d1b, k2, d2W, d2b, k3, d3W, d3b):
    px, py, pz = pos[:, 0], pos[:, 1], pos[:, 2]
    bx, by, bz = box[:, 0], box[:, 1], box[:, 2]
    ep = EPAD - E
    srcf = jnp.pad(edge_index_fluid[0], (0, ep))
    dstf = jnp.pad(edge_index_fluid[1], (0, ep))
    srco = jnp.pad(edge_index_obstacle[0], (0, ep))
    dsto = jnp.pad(edge_index_obstacle[1], (0, ep))
    wf8, if8, dcf, ccf = _geom(px, py, pz, px, py, pz, srcf, dstf)
    wo8, io8, dco, cco = _geom(bx, by, bz, px, py, pz, srco, dsto)

    fluid_feats = jnp.concatenate([jnp.ones_like(pos[:, 0:1]), vel], axis=-1)
    ff_pad = jnp.pad(fluid_feats, ((0, 0), (0, 4)))
    bf_pad = jnp.pad(box_feats, ((0, 0), (0, 5)))

    def conv(x_pad, kflat_mat, w8, i8, dst_h, cnt_h, cout, relu):
        cgs = 32
        p = _conv(_mm(x_pad, kflat_mat, relu=relu).reshape(N * 64, cout),
                  w8, i8, dst_h.reshape(NSUB * EPSR // cgs, cgs), cnt_h,
                  cout, cgs)
        return p[0, :N], p[1, :N]

    k0f_f = _kflat(jnp.pad(k0f, ((0, 0),) * 3 + ((0, 4), (0, 0))))
    k0o_f = _kflat(jnp.pad(k0o, ((0, 0),) * 3 + ((0, 5), (0, 0))))
    c0f = _addk(conv(ff_pad, k0f_f, wf8, if8, dcf, ccf, 32, False))
    c0o = _addk(conv(bf_pad, k0o_f, wo8, io8, dco, cco, 32, False))
    d0 = _mm(ff_pad, jnp.pad(d0W, ((0, 4), (0, 0))), d0b, False)
    feats = jnp.concatenate([c0o, c0f, d0], axis=-1)

    p0, p1 = conv(feats, _kflat(k1), wf8, if8, dcf, ccf, 64, True)
    d1 = _mm(feats, d1W, d1b, True)
    ans1 = _addk((p0, p1, d1))
    p0, p1 = conv(ans1, _kflat(k2), wf8, if8, dcf, ccf, 64, True)
    d2 = _mm(ans1, d2W, d2b, True)
    ans2 = _addk((p0, p1, d2, ans1))
    p0, p1 = conv(ans2, _kflat(k3, cout_pad=16), wf8, if8, dcf, ccf, 16, True)
    d3 = _mm(ans2, jnp.pad(d3W, ((0, 0), (0, 13))),
             jnp.pad(d3b, (0, 13)), True)
    ans3 = _addk((p0, p1, d3), scale=1.0 / 128.0)
    return ans3[:, :3]


# speedup vs baseline: 1.2675x; 1.0272x over previous
import functools

import jax
import jax.numpy as jnp
from jax import lax
from jax.experimental import pallas as pl
from jax.experimental.pallas import tpu as pltpu
from jax.experimental.pallas import tpu_sc as plsc
import numpy as np

N = 10000
E = 160000
EXTENT = np.float32(1.5 * 6 * 0.025)
NSUB = 32
SB = 1024
NSTG = 5
EPS = SB * NSTG
EPAD = NSUB * EPS
CG = 64
SBC = 512
NPAIR = SBC // (2 * CG)
EPSR = EPS + SB
NSTGR = EPSR // SBC
NPAD = 10240
RPS = NPAD // 16

_MESH = dict(mesh=plsc.VectorSubcoreMesh(core_axis_name="c", subcore_axis_name="s"),
             compiler_params=pltpu.CompilerParams(needs_layout_passes=False,
                                                  use_tc_tiling_on_sc=False))

def _wid():
    return lax.axis_index("s") * 2 + lax.axis_index("c")


def _splat_i32(v):
    return (jnp.broadcast_to(jnp.int32(v), (16,)) if isinstance(v, int)
            else lax.broadcast(v, (16,)))


def _sqrt16(x):
    i = plsc.bitcast(x, jnp.int32)
    y = plsc.bitcast(lax.shift_right_logical(i, 1) + jnp.int32(0x1FBD1DF5), jnp.float32)
    for _ in range(3):
        y = 0.5 * (y + x / y)
    return y


def _geom_body(sx_h, sy_h, sz_h, dx_h, dy_h, dz_h, src_h, dst_h,
               w_out, i_out, d_out, c_out,
               sx, sy, sz, dx, dy, dz, srcv, dstv, wbuf, ibuf, dbuf, cntbuf):
    wid = _wid()
    pltpu.sync_copy(sx_h, sx)
    pltpu.sync_copy(sy_h, sy)
    pltpu.sync_copy(sz_h, sz)
    pltpu.sync_copy(dx_h, dx)
    pltpu.sync_copy(dy_h, dy)
    pltpu.sync_copy(dz_h, dz)
    scale = jnp.float32(2.0 / EXTENT)
    zf = jnp.zeros((16,), jnp.float32)
    zi = jnp.zeros((16,), jnp.int32)
    cnt = jnp.int32(0)

    for kc in range(NSTG):
        base = wid * EPS + kc * SB
        pltpu.sync_copy(src_h.at[pl.ds(base, SB)], srcv)
        pltpu.sync_copy(dst_h.at[pl.ds(base, SB)], dstv)

        def zrow(r, _):
            for c in range(8):
                wbuf[c, pl.ds(r * 16, 16)] = zf
                ibuf[c, pl.ds(r * 16, 16)] = zi
            dbuf[pl.ds(r * 16, 16)] = zi
            return _
        lax.fori_loop(0, (SB + 16) // 16, zrow, None)

        def grp(g, off):
            s16 = srcv[pl.ds(g * 16, 16)]
            d16 = dstv[pl.ds(g * 16, 16)]
            xs = plsc.load_gather(sx, [s16])
            ys = plsc.load_gather(sy, [s16])
            zs = plsc.load_gather(sz, [s16])
            xd = plsc.load_gather(dx, [d16])
            yd = plsc.load_gather(dy, [d16])
            zd = plsc.load_gather(dz, [d16])
            rx = (xs - xd) * scale
            ry = (ys - yd) * scale
            rz = (zs - zd) * scale
            r2 = rx * rx + ry * ry + rz * rz
            t = 1.0 - r2
            win = jnp.clip(t * t * t, 0.0, 1.0)
            eid = base + g * 16 + lax.iota(jnp.int32, 16)
            win = jnp.where(eid < E, win, 0.0)
            mask = win > 0.0
            l2 = _sqrt16(jnp.maximum(r2, 1e-12))
            linf = jnp.maximum(
                jnp.maximum(jnp.maximum(jnp.abs(rx), jnp.abs(ry)), jnp.abs(rz)),
                1e-12)
            q = l2 / linf
            ax = jnp.clip(rx * q, -1.0, 1.0)
            ay = jnp.clip(ry * q, -1.0, 1.0)
            az = jnp.clip(rz * q, -1.0, 1.0)
            ux = (ax + 1.0) * 1.5
            uy = (ay + 1.0) * 1.5
            uz = (az + 1.0) * 1.5
            lx = jnp.minimum(ux.astype(jnp.int32), 2)
            ly = jnp.minimum(uy.astype(jnp.int32), 2)
            lz = jnp.minimum(uz.astype(jnp.int32), 2)
            fx = ux - lx.astype(jnp.float32)
            fy = uy - ly.astype(jnp.float32)
            fz = uz - lz.astype(jnp.float32)
            base_i = s16 * 64 + lx * 16 + ly * 4 + lz
            for c in range(8):
                ox, oy, oz = (c >> 2) & 1, (c >> 1) & 1, c & 1
                w = ((fx if ox else 1.0 - fx)
                     * (fy if oy else 1.0 - fy)
                     * (fz if oz else 1.0 - fz) * win)
                gi = base_i + (ox * 16 + oy * 4 + oz)
                plsc.store_compressed(wbuf.at[c, pl.ds(off, 16)], w, mask=mask)
                plsc.store_compressed(ibuf.at[c, pl.ds(off, 16)], gi, mask=mask)
            plsc.store_compressed(dbuf.at[pl.ds(off, 16)], d16, mask=mask)
            return off + jnp.sum(jnp.where(mask, 1, 0).astype(jnp.int32))

        off = lax.fori_loop(0, SB // 16, grp, jnp.int32(0))
        cnt = pl.multiple_of(cnt, 8)
        pltpu.sync_copy(wbuf.at[:, pl.ds(0, SB)],
                        w_out.at[:, pl.ds(wid * EPSR + cnt, SB)])
        pltpu.sync_copy(ibuf.at[:, pl.ds(0, SB)],
                        i_out.at[:, pl.ds(wid * EPSR + cnt, SB)])
        pltpu.sync_copy(dbuf.at[pl.ds(0, SB)],
                        d_out.at[pl.ds(wid * EPSR + cnt, SB)])
        cnt = cnt + jnp.bitwise_and(off + 7, jnp.int32(-8))

    cnt = pl.multiple_of(cnt, 8)
    def zrow2(r, _):
        for c in range(8):
            wbuf[c, pl.ds(r * 16, 16)] = zf
            ibuf[c, pl.ds(r * 16, 16)] = zi
        dbuf[pl.ds(r * 16, 16)] = zi
        return _
    lax.fori_loop(0, (SB + 16) // 16, zrow2, None)
    pltpu.sync_copy(wbuf.at[:, pl.ds(0, SB)],
                    w_out.at[:, pl.ds(wid * EPSR + cnt, SB)])
    pltpu.sync_copy(ibuf.at[:, pl.ds(0, SB)],
                    i_out.at[:, pl.ds(wid * EPSR + cnt, SB)])
    pltpu.sync_copy(dbuf.at[pl.ds(0, SB)],
                    d_out.at[pl.ds(wid * EPSR + cnt, SB)])

    cntbuf[pl.ds(0, 16)] = lax.broadcast(cnt, (16,))
    pltpu.sync_copy(cntbuf.at[pl.ds(0, 8)], c_out.at[wid])


_geom = functools.partial(
    pl.kernel,
    out_type=[jax.ShapeDtypeStruct((8, NSUB * EPSR), jnp.float32),
              jax.ShapeDtypeStruct((8, NSUB * EPSR), jnp.int32),
              jax.ShapeDtypeStruct((NSUB * EPSR,), jnp.int32),
              jax.ShapeDtypeStruct((NSUB, 8), jnp.int32)],
    scratch_types=[pltpu.VMEM((N,), jnp.float32)] * 6
    + [pltpu.VMEM((SB,), jnp.int32)] * 2
    + [pltpu.VMEM((8, SB + 16), jnp.float32), pltpu.VMEM((8, SB + 16), jnp.int32),
       pltpu.VMEM((SB + 16,), jnp.int32), pltpu.VMEM((16,), jnp.int32)],
    **_MESH,
)(_geom_body)


def _conv_body(cout, cgs, y, w8, i8, dst_h, cnt_h, out,
               wv, iv, dstv, cntv, rows_a, rows_b, acc_a, acc_b, zbuf, out_sh,
               sem_a, sem_b, sem_sa, sem_sb):
    wid = _wid()
    scid = lax.axis_index("c")
    sid = lax.axis_index("s")
    nq = cout // 16

    def zrow(r, _):
        for qq in range(nq):
            zbuf[r, pl.ds(qq * 16, 16)] = jnp.zeros((16,), jnp.float32)
        return _
    lax.fori_loop(0, 64, zrow, None)
    for j in range(10):
        pltpu.sync_copy(zbuf, out_sh.at[pl.ds(sid * RPS + j * 64, 64)])
    plsc.subcore_barrier()

    def fire(cg, rows, sem):
        for c in range(8):
            pltpu.async_copy(y.at[iv.at[c, pl.ds(cg * cgs, cgs)]], rows.at[c], sem)

    def drain(cg, rows, sem):
        for c in range(8):
            pltpu.make_async_copy(y.at[iv.at[c, pl.ds(cg * cgs, cgs)]],
                                  rows.at[c], sem).wait()

    def compute(cg, rows, acc):
        def grp(g, _):
            e0 = g * 16
            for j in range(16):
                e = e0 + j
                ev = _splat_i32(cg * cgs + e)
                av = [None] * nq
                for c in range(8):
                    ws = plsc.load_gather(wv, [_splat_i32(c), ev])
                    for qq in range(nq):
                        term = ws * rows[c, e, pl.ds(qq * 16, 16)]
                        av[qq] = term if c == 0 else av[qq] + term
                for qq in range(nq):
                    acc[e, pl.ds(qq * 16, 16)] = av[qq]
            return _
        lax.fori_loop(0, cgs // 16, grp, None)

    def scatter_fire(cg, acc, sem):
        pltpu.async_copy(acc, out_sh.at[dstv.at[cg]], sem, add=True)

    def scatter_wait(acc, sem):
        pltpu.make_async_copy(acc, out_sh.at[dstv.at[0]], sem).wait()

    pltpu.sync_copy(cnt_h.at[wid], cntv.at[pl.ds(0, 8)])
    cnt = cntv[pl.ds(0, 16)][0]
    nst = (cnt + (SBC - 1)) // SBC

    def stage(s, _):
        base = wid * EPSR + s * SBC
        sb0 = s * SBC
        pltpu.sync_copy(w8.at[:, pl.ds(base, SBC)], wv)
        pltpu.sync_copy(i8.at[:, pl.ds(base, SBC)], iv)
        pltpu.sync_copy(dst_h.at[pl.ds(base // cgs, SBC // cgs)], dstv)
        fire(0, rows_a, sem_a)

        def pair(p, _):
            ca = 2 * p
            cb = 2 * p + 1
            act_a = sb0 + ca * cgs < cnt
            act_b = sb0 + cb * cgs < cnt

            @pl.when(act_b)
            def _():
                fire(cb, rows_b, sem_b)

            @pl.when(jnp.logical_and(act_a, p > 0))
            def _():
                scatter_wait(acc_a, sem_sa)

            @pl.when(act_a)
            def _():
                drain(ca, rows_a, sem_a)
                compute(ca, rows_a, acc_a)
                scatter_fire(ca, acc_a, sem_sa)

            @pl.when(jnp.logical_and(p < SBC // (2 * cgs) - 1, sb0 + (ca + 2) * cgs < cnt))
            def _():
                fire(ca + 2, rows_a, sem_a)

            @pl.when(jnp.logical_and(act_b, p > 0))
            def _():
                scatter_wait(acc_b, sem_sb)

            @pl.when(act_b)
            def _():
                drain(cb, rows_b, sem_b)
                compute(cb, rows_b, acc_b)
                scatter_fire(cb, acc_b, sem_sb)
            return _

        lax.fori_loop(0, SBC // (2 * cgs), pair, None)
        scatter_wait(acc_a, sem_sa)

        @pl.when(sb0 + cgs < cnt)
        def _():
            scatter_wait(acc_b, sem_sb)
        return _

    lax.fori_loop(0, nst, stage, None)
    plsc.subcore_barrier()
    for j in range(5):
        r0 = sid * RPS + j * 128
        pltpu.sync_copy(out_sh.at[pl.ds(r0, 128)], out.at[scid, pl.ds(r0, 128)])


def _conv(y, w8, i8, dst_h, cnt_h, cout, cgs):
    body = functools.partial(_conv_body, cout, cgs)
    return pl.kernel(
        body,
        out_type=jax.ShapeDtypeStruct((2, NPAD, cout), jnp.float32),
        scratch_types=[
            pltpu.VMEM((8, SBC), jnp.float32),
            pltpu.VMEM((8, SBC), jnp.int32),
            pltpu.VMEM((SBC // cgs, cgs), jnp.int32),
            pltpu.VMEM((16,), jnp.int32),
            pltpu.VMEM((8, cgs, cout), jnp.float32),
            pltpu.VMEM((8, cgs, cout), jnp.float32),
            pltpu.VMEM((cgs, cout), jnp.float32),
            pltpu.VMEM((cgs, cout), jnp.float32),
            pltpu.VMEM((64, cout), jnp.float32),
            pltpu.VMEM_SHARED((NPAD, cout), jnp.float32),
            pltpu.SemaphoreType.DMA,
            pltpu.SemaphoreType.DMA,
            pltpu.SemaphoreType.DMA,
            pltpu.SemaphoreType.DMA,
        ],
        **_MESH,
    )(y, w8, i8, dst_h, cnt_h)


def _mm(x, w, b=None, relu=False):
    m, kin = x.shape
    kout = w.shape[1]
    bn = 1000
    bk = min(kout, 512)

    def body(*refs):
        x_ref, w_ref = refs[0], refs[1]
        o_ref = refs[-1]
        xx = x_ref[...]
        if relu:
            xx = jnp.maximum(xx, 0.0)
        o = jnp.dot(xx, w_ref[...], preferred_element_type=jnp.float32)
        if b is not None:
            o = o + refs[2][...]
        o_ref[...] = o

    in_specs = [pl.BlockSpec((bn, kin), lambda i, j: (i, 0)),
                pl.BlockSpec((kin, bk), lambda i, j: (0, j))]
    args = [x, w]
    if b is not None:
        in_specs.append(pl.BlockSpec((1, bk), lambda i, j: (0, j)))
        args.append(b.reshape(1, kout))
    return pl.pallas_call(
        body,
        grid=(m // bn, kout // bk),
        in_specs=in_specs,
        out_specs=pl.BlockSpec((bn, bk), lambda i, j: (i, j)),
        out_shape=jax.ShapeDtypeStruct((m, kout), jnp.float32),
    )(*args)


def _addk(arrays, scale=1.0):
    m, c = arrays[0].shape
    bn = 1000

    def body(*refs):
        o_ref = refs[-1]
        s = refs[0][...]
        for r in refs[1:-1]:
            s = s + r[...]
        o_ref[...] = s * scale

    return pl.pallas_call(
        body,
        grid=(m // bn,),
        in_specs=[pl.BlockSpec((bn, c), lambda i: (i, 0)) for _ in arrays],
        out_specs=pl.BlockSpec((bn, c), lambda i: (i, 0)),
        out_shape=jax.ShapeDtypeStruct((m, c), jnp.float32),
    )(*arrays)


def _kflat(kk, cout_pad=None):
    cin, cout = kk.shape[3], kk.shape[4]
    k = kk.reshape(64, cin, cout)
    if cout_pad is not None and cout_pad != cout:
        k = jnp.pad(k, ((0, 0), (0, 0), (0, cout_pad - cout)))
        cout = cout_pad
    return k.transpose(1, 0, 2).reshape(cin, 64 * cout)


def kernel(pos, vel, box, box_feats, edge_index_fluid, edge_index_obstacle,
           k0f, k0o, d0W, d0b, k1, d1W, d1b, k2, d2W, d2b, k3, d3W, d3b):
    px, py, pz = pos[:, 0], pos[:, 1], pos[:, 2]
    bx, by, bz = box[:, 0], box[:, 1], box[:, 2]
    ep = EPAD - E
    srcf = jnp.pad(edge_index_fluid[0], (0, ep))
    dstf = jnp.pad(edge_index_fluid[1], (0, ep))
    srco = jnp.pad(edge_index_obstacle[0], (0, ep))
    dsto = jnp.pad(edge_index_obstacle[1], (0, ep))
    wf8, if8, dcf, ccf = _geom(px, py, pz, px, py, pz, srcf, dstf)
    wo8, io8, dco, cco = _geom(bx, by, bz, px, py, pz, srco, dsto)

    fluid_feats = jnp.concatenate([jnp.ones_like(pos[:, 0:1]), vel], axis=-1)
    ff_pad = jnp.pad(fluid_feats, ((0, 0), (0, 4)))
    bf_pad = jnp.pad(box_feats, ((0, 0), (0, 5)))

    def conv(x_pad, kflat_mat, w8, i8, dst_h, cnt_h, cout, relu):
        cgs = 16
        p = _conv(_mm(x_pad, kflat_mat, relu=relu).reshape(N * 64, cout),
                  w8, i8, dst_h.reshape(NSUB * EPSR // cgs, cgs), cnt_h,
                  cout, cgs)
        return p[0, :N], p[1, :N]

    k0f_f = _kflat(jnp.pad(k0f, ((0, 0),) * 3 + ((0, 4), (0, 0))))
    k0o_f = _kflat(jnp.pad(k0o, ((0, 0),) * 3 + ((0, 5), (0, 0))))
    c0f = _addk(conv(ff_pad, k0f_f, wf8, if8, dcf, ccf, 32, False))
    c0o = _addk(conv(bf_pad, k0o_f, wo8, io8, dco, cco, 32, False))
    d0 = _mm(ff_pad, jnp.pad(d0W, ((0, 4), (0, 0))), d0b, False)
    feats = jnp.concatenate([c0o, c0f, d0], axis=-1)

    p0, p1 = conv(feats, _kflat(k1), wf8, if8, dcf, ccf, 64, True)
    d1 = _mm(feats, d1W, d1b, True)
    ans1 = _addk((p0, p1, d1))
    p0, p1 = conv(ans1, _kflat(k2), wf8, if8, dcf, ccf, 64, True)
    d2 = _mm(ans1, d2W, d2b, True)
    ans2 = _addk((p0, p1, d2, ans1))
    p0, p1 = conv(ans2, _kflat(k3, cout_pad=16), wf8, if8, dcf, ccf, 16, True)
    d3 = _mm(ans2, jnp.pad(d3W, ((0, 0), (0, 13))),
             jnp.pad(d3b, (0, 13)), True)
    ans3 = _addk((p0, p1, d3), scale=1.0 / 128.0)
    return ans3[:, :3]
